# Initial kernel scaffold; baseline (speedup 1.0000x reference)
import jax
import jax.numpy as jnp
from jax.experimental import pallas as pl


def kernel(x, W1, b1, W2, b2, cls, Wq, Wk, Wv, Wo, Wout):
    raise NotImplementedError("write your pallas kernel here")


# speedup vs baseline: 2.4172x; 2.4172x over previous
import jax
import jax.numpy as jnp
import numpy as np
from jax.experimental import pallas as pl
from jax.experimental.pallas import tpu as pltpu

_P = 32
_K = 8
_TOK = 64
_HID = 128
_DOUT = 256
_NH = 4
_HD = _TOK // _NH


def _part1by2(n):
    n = n & 0x3FF
    n = (n | (n << 16)) & 0x030000FF
    n = (n | (n << 8)) & 0x0300F00F
    n = (n | (n << 4)) & 0x030C30C3
    n = (n | (n << 2)) & 0x09249249
    return n


def _pointnet_kernel(x_ref, W1c_ref, b1_ref, W2_ref, b2_ref, cls_ref,
                     Wq_ref, Wk_ref, Wv_ref, Wo_ref, Wout_ref, out_ref):
    x = x_ref[0]
    N = x.shape[0]
    S = N // _P

    mn = jnp.min(x, axis=0, keepdims=True)
    mx = jnp.max(x, axis=0, keepdims=True)
    g = jnp.clip((x - mn) / (mx - mn + 1e-9) * 1023.0, 0.0, 1023.0)
    gi = g.astype(jnp.int32)
    c0 = _part1by2(gi[:, 0:1])
    c1 = _part1by2(gi[:, 1:2])
    c2 = _part1by2(gi[:, 2:3])
    code = (c0 << 2) | (c1 << 1) | c2

    code_row = code.reshape(1, N)
    ii = jax.lax.broadcasted_iota(jnp.int32, (N, N), 0)
    jj = jax.lax.broadcasted_iota(jnp.int32, (N, N), 1)
    lt = (code < code_row)
    eq = (code == code_row)
    rank = jnp.sum(lt.astype(jnp.int32) + (eq & (ii < jj)).astype(jnp.int32),
                   axis=0).reshape(1, N)
    oh = (rank == ii).astype(jnp.float32)
    xs = jnp.dot(oh, x, preferred_element_type=jnp.float32)

    AB = jnp.dot(xs, W1c_ref[...], preferred_element_type=jnp.float32)
    A = AB[:, :_HID]
    Bv = AB[:, _HID:]
    b1 = b1_ref[...].reshape(1, _HID)
    b2 = b2_ref[...].reshape(1, _TOK)
    W2 = W2_ref[...]

    toks = []
    qq = jax.lax.broadcasted_iota(jnp.int32, (_P, _P), 1)
    inf = jnp.float32(jnp.inf)
    for s in range(S):
        xp = xs[s * _P:(s + 1) * _P]
        sq = jnp.sum(xp * xp, axis=1, keepdims=True)
        gram = jax.lax.dot_general(
            xp, xp, (((1,), (1,)), ((), ())),
            preferred_element_type=jnp.float32)
        d2 = sq + sq.reshape(1, _P) - 2.0 * gram
        d2w = d2
        ohs = []
        for _ in range(_K):
            rowmin = jnp.min(d2w, axis=1, keepdims=True)
            eqm = d2w == rowmin
            am = jnp.min(jnp.where(eqm, qq, _P), axis=1, keepdims=True)
            first = qq == am
            ohs.append(first.astype(jnp.float32))
            d2w = jnp.where(first, inf, d2w)
        OH = jnp.concatenate(ohs, axis=0)
        Bp = Bv[s * _P:(s + 1) * _P]
        G = jnp.dot(OH, Bp, preferred_element_type=jnp.float32)
        M = G[0:_P]
        for k in range(1, _K):
            M = jnp.maximum(M, G[k * _P:(k + 1) * _P])
        m = jnp.maximum(A[s * _P:(s + 1) * _P] + M + b1, 0.0)
        h2 = jnp.maximum(
            jnp.dot(m, W2, preferred_element_type=jnp.float32) + b2, 0.0)
        toks.append(jnp.max(h2, axis=0, keepdims=True))

    tok = jnp.concatenate(toks, axis=0)

    t = jnp.concatenate([cls_ref[...].reshape(1, _TOK), tok], axis=0)
    kk = jnp.dot(t, Wk_ref[...], preferred_element_type=jnp.float32)
    vv = jnp.dot(t, Wv_ref[...], preferred_element_type=jnp.float32)
    q0 = jnp.dot(t[0:1], Wq_ref[...], preferred_element_type=jnp.float32)
    heads = []
    for h in range(_NH):
        qh = q0[:, h * _HD:(h + 1) * _HD]
        kh = kk[:, h * _HD:(h + 1) * _HD]
        vh = vv[:, h * _HD:(h + 1) * _HD]
        sc = jax.lax.dot_general(
            qh, kh, (((1,), (1,)), ((), ())),
            preferred_element_type=jnp.float32) / np.float32(np.sqrt(_HD))
        aw = jax.nn.softmax(sc, axis=-1)
        heads.append(jnp.dot(aw, vh, preferred_element_type=jnp.float32))
    o0 = jnp.concatenate(heads, axis=1)
    o0 = jnp.dot(o0, Wo_ref[...], preferred_element_type=jnp.float32) + t[0:1]
    out_ref[...] = jnp.dot(o0, Wout_ref[...],
                           preferred_element_type=jnp.float32).reshape(1, 1, _DOUT)


def kernel(x, W1, b1, W2, b2, cls, Wq, Wk, Wv, Wo, Wout):
    B, N, D = x.shape
    W1a = W1[:D]
    W1b = W1[D:]
    W1c = jnp.concatenate([W1a - W1b, W1b], axis=1)

    grid = (B,)
    out = pl.pallas_call(
        _pointnet_kernel,
        grid=grid,
        in_specs=[
            pl.BlockSpec((1, N, D), lambda i: (i, 0, 0)),
            pl.BlockSpec((D, 2 * _HID), lambda i: (0, 0)),
            pl.BlockSpec((_HID,), lambda i: (0,)),
            pl.BlockSpec((_HID, _TOK), lambda i: (0, 0)),
            pl.BlockSpec((_TOK,), lambda i: (0,)),
            pl.BlockSpec((_TOK,), lambda i: (0,)),
            pl.BlockSpec((_TOK, _TOK), lambda i: (0, 0)),
            pl.BlockSpec((_TOK, _TOK), lambda i: (0, 0)),
            pl.BlockSpec((_TOK, _TOK), lambda i: (0, 0)),
            pl.BlockSpec((_TOK, _TOK), lambda i: (0, 0)),
            pl.BlockSpec((_TOK, _DOUT), lambda i: (0, 0)),
        ],
        out_specs=pl.BlockSpec((1, 1, _DOUT), lambda i: (i, 0, 0)),
        out_shape=jax.ShapeDtypeStruct((B, 1, _DOUT), jnp.float32),
        compiler_params=pltpu.CompilerParams(
            dimension_semantics=("arbitrary",)),
    )(x, W1c, b1, W2, b2, cls, Wq, Wk, Wv, Wo, Wout)
    return out.reshape(B, _DOUT)
